# baseline (speedup 1.0000x reference)
import functools

import jax
import jax.numpy as jnp
from jax import lax
from jax.experimental import pallas as pl
from jax.experimental.pallas import tpu as pltpu

_K = 7
_PAD = 3


def _pool_chunk(xc, padm_ref, pada_ref, *, n_tile, C, HW, LPAD, inv_c):
    for t in range(n_tile):
        acc_m = xc[t, 0:8, :]
        acc_s = acc_m
        for r in range(8, C - (C % 8), 8):
            blk = xc[t, r:r + 8, :]
            acc_m = jnp.maximum(acc_m, blk)
            acc_s = acc_s + blk
        if C % 8:
            blk = xc[t, C - (C % 8):C, :]
            acc_m = jnp.maximum(acc_m, jnp.max(blk, axis=0, keepdims=True))
            acc_s = acc_s + jnp.sum(blk, axis=0, keepdims=True)
        m = jnp.max(acc_m, axis=0, keepdims=True)
        s = jnp.sum(acc_s, axis=0, keepdims=True)
        padm_ref[t:t + 1, LPAD:LPAD + HW] = m
        pada_ref[t:t + 1, LPAD:LPAD + HW] = s * inv_c


def _conv_sigmoid(wv, bv, mask_ref, padm_ref, pada_ref, *, n_tile, W, HW,
                  LPAD):
    acc = jnp.zeros((n_tile, HW), dtype=jnp.float32)
    for dx in range(_K):
        pm = jnp.zeros((n_tile, HW), dtype=jnp.float32)
        pa = jnp.zeros((n_tile, HW), dtype=jnp.float32)
        for dy in range(_K):
            off = LPAD + (dy - _PAD) * W + (dx - _PAD)
            pm = pm + wv[dy * _K + dx] * padm_ref[:, off:off + HW]
            pa = pa + wv[_K * _K + dy * _K + dx] * pada_ref[:, off:off + HW]
        acc = acc + (pm + pa) * mask_ref[dx:dx + 1, :]
    return jax.nn.sigmoid(acc + bv)


def _sa_body(w_ref, b_ref, mask_ref, x_hbm, o_ref, buf, padm_ref, pada_ref,
             sems, *, n_tile, steps, depth, C, W, HW, LPAD, inv_c):
    core = pl.program_id(0)
    img0 = core * (steps * n_tile)

    def start_in(i, slot):
        pltpu.make_async_copy(
            x_hbm.at[pl.ds(img0 + i * n_tile, n_tile)],
            buf.at[slot],
            sems.at[slot]).start()

    def wait_in(slot):
        pltpu.make_async_copy(
            x_hbm.at[pl.ds(0, n_tile)],
            buf.at[slot],
            sems.at[slot]).wait()

    zpad = jnp.zeros((n_tile, LPAD), dtype=jnp.float32)
    padm_ref[:, :LPAD] = zpad
    padm_ref[:, LPAD + HW:] = zpad
    pada_ref[:, :LPAD] = zpad
    pada_ref[:, LPAD + HW:] = zpad

    wv = [w_ref[i] for i in range(2 * _K * _K)]
    bv = b_ref[0]

    for i in range(min(depth - 1, steps)):
        start_in(i, i)

    def step_fn(i, carry):
        slot = lax.rem(i, depth)

        @pl.when(i + depth - 1 < steps)
        def _():
            start_in(i + depth - 1, lax.rem(i + depth - 1, depth))

        wait_in(slot)
        xc = buf.at[slot]
        _pool_chunk(xc, padm_ref, pada_ref, n_tile=n_tile, C=C, HW=HW,
                    LPAD=LPAD, inv_c=inv_c)
        out = _conv_sigmoid(wv, bv, mask_ref, padm_ref, pada_ref,
                            n_tile=n_tile, W=W, HW=HW, LPAD=LPAD)
        o_ref[pl.ds(i * n_tile, n_tile), 0, :] = out.astype(o_ref.dtype)
        return carry

    lax.fori_loop(0, steps, step_fn, 0)


def kernel(x, weight, bias):
    N, C, H, W = x.shape
    HW = H * W
    itemsize = jnp.dtype(x.dtype).itemsize

    n_cores = 2 if N % 2 == 0 else 1
    n_tile = 1
    for t in (4, 2):
        if (N // n_cores) % t == 0:
            n_tile = t
            break
    steps = N // (n_cores * n_tile)
    depth = min(6, steps)

    LPAD = ((_PAD * (W + 1) + 127) // 128) * 128
    Wpad = 2 * LPAD + HW

    x_flat = x.reshape(N, C, HW)
    w_flat = weight.reshape(-1).astype(jnp.float32)
    b = bias.astype(jnp.float32)

    cols = jnp.tile(jnp.arange(W, dtype=jnp.int32), H)
    dxs = jnp.arange(_K, dtype=jnp.int32)[:, None]
    colmask = ((cols[None, :] + dxs - _PAD >= 0)
               & (cols[None, :] + dxs - _PAD < W)).astype(jnp.float32)

    body = functools.partial(_sa_body, n_tile=n_tile, steps=steps,
                             depth=depth, C=C, W=W, HW=HW, LPAD=LPAD,
                             inv_c=1.0 / float(C))

    cost = pl.CostEstimate(
        flops=int(N * HW * (2 * C + 4 * _K * _K + _K)),
        transcendentals=int(N * HW),
        bytes_accessed=int(N * C * HW * itemsize + N * HW * itemsize
                           + _K * HW * 4 + (2 * _K * _K + 1) * 4),
    )

    out = pl.pallas_call(
        body,
        out_shape=jax.ShapeDtypeStruct((N, 1, HW), x.dtype),
        grid=(n_cores,),
        in_specs=[
            pl.BlockSpec(memory_space=pltpu.SMEM),
            pl.BlockSpec(memory_space=pltpu.SMEM),
            pl.BlockSpec((_K, HW), lambda c: (0, 0)),
            pl.BlockSpec(memory_space=pl.ANY),
        ],
        out_specs=pl.BlockSpec((N // n_cores, 1, HW), lambda c: (c, 0, 0)),
        scratch_shapes=[
            pltpu.VMEM((depth, n_tile, C, HW), x.dtype),
            pltpu.VMEM((n_tile, Wpad), jnp.float32),
            pltpu.VMEM((n_tile, Wpad), jnp.float32),
            pltpu.SemaphoreType.DMA((depth,)),
        ],
        compiler_params=pltpu.CompilerParams(
            dimension_semantics=("parallel",)),
        cost_estimate=cost,
    )(w_flat, b, colmask, x_flat)

    return out.reshape(N, 1, H, W)
